# baseline (speedup 1.0000x reference)
import functools

import jax
import jax.numpy as jnp
from jax import lax
from jax.experimental import pallas as pl
from jax.experimental.pallas import tpu as pltpu
from jax.experimental.pallas import tpu_sc as plsc

F = 128
S = 10


def _silu(x):
    return x * (1.0 / (1.0 + jnp.exp(-x)))


def _node_pre_body(nf_ref, na_ref, wup_ref, wskipT_ref, h0q_ref, skip0_ref):
    x = nf_ref[...]
    h0q_ref[...] = jnp.dot(x, wup_ref[...], preferred_element_type=jnp.float32) * (
        F ** -0.5
    )
    a = na_ref[...]
    acc = jnp.zeros(x.shape, jnp.float32)
    for v in range(S):
        acc = acc + a[:, v : v + 1] * jnp.dot(
            x, wskipT_ref[v], preferred_element_type=jnp.float32
        )
    skip0_ref[...] = acc * ((F * S) ** -0.5)


def _node_pre(node_feats, node_attrs, W_up, W_skipT, *, interpret=False):
    N = node_feats.shape[0]
    NB = 1000
    return pl.pallas_call(
        _node_pre_body,
        grid=(N // NB,),
        in_specs=[
            pl.BlockSpec((NB, F), lambda i: (i, 0)),
            pl.BlockSpec((NB, S), lambda i: (i, 0)),
            pl.BlockSpec((F, F), lambda i: (0, 0)),
            pl.BlockSpec((S, F, F), lambda i: (0, 0, 0)),
        ],
        out_specs=[
            pl.BlockSpec((NB, F), lambda i: (i, 0)),
            pl.BlockSpec((NB, F), lambda i: (i, 0)),
        ],
        out_shape=[
            jax.ShapeDtypeStruct((N, F), jnp.float32),
            jax.ShapeDtypeStruct((N, F), jnp.float32),
        ],
        interpret=interpret,
    )(node_feats, node_attrs, W_up, W_skipT)


def _edge_pre_body(ef_ref, ea_ref, wr1_ref, wr2_ref, wr3_ref, wr4_ref,
                   cw0_ref, cw1_ref):
    x = ef_ref[...]
    w = _silu(jnp.dot(x, wr1_ref[...], preferred_element_type=jnp.float32) * (8.0 ** -0.5))
    w = _silu(jnp.dot(w, wr2_ref[...], preferred_element_type=jnp.float32) * (64.0 ** -0.5))
    w = _silu(jnp.dot(w, wr3_ref[...], preferred_element_type=jnp.float32) * (64.0 ** -0.5))
    w = jnp.dot(w, wr4_ref[...], preferred_element_type=jnp.float32) * (64.0 ** -0.5)
    y0 = ea_ref[...][:, 0:1]
    for q in range(4):
        cw0_ref[q] = w[:, 32 * q : 32 * (q + 1)] * y0
        cw1_ref[q] = w[:, F + 32 * q : F + 32 * (q + 1)]


def _edge_pre(edge_feats, edge_attrs, W_r1, W_r2, W_r3, W_r4b, *, interpret=False):
    E = edge_feats.shape[0]
    EB = 2000
    return pl.pallas_call(
        _edge_pre_body,
        grid=(E // EB,),
        in_specs=[
            pl.BlockSpec((EB, 8), lambda i: (i, 0)),
            pl.BlockSpec((EB, 4), lambda i: (i, 0)),
            pl.BlockSpec((8, 64), lambda i: (0, 0)),
            pl.BlockSpec((64, 64), lambda i: (0, 0)),
            pl.BlockSpec((64, 64), lambda i: (0, 0)),
            pl.BlockSpec((64, 2 * F), lambda i: (0, 0)),
        ],
        out_specs=[
            pl.BlockSpec((4, EB, 32), lambda i: (0, i, 0)),
            pl.BlockSpec((4, EB, 32), lambda i: (0, i, 0)),
        ],
        out_shape=[
            jax.ShapeDtypeStruct((4, E, 32), jnp.float32),
            jax.ShapeDtypeStruct((4, E, 32), jnp.float32),
        ],
        interpret=interpret,
    )(edge_feats, edge_attrs, W_r1, W_r2, W_r3, W_r4b)


def _node_post_body(
    part_ref, na_ref, wl0_ref, wl1_ref, wm0T_ref, wm1T_ref, wo0_ref, wo1_ref,
    f0_ref, fx_ref, fy_ref, fz_ref,
):
    c1 = ((2 * F) ** -0.5) * 0.25
    c2 = (F * S) ** -0.5
    c3 = F ** -0.5
    a = na_ref[...]
    psum = [part_ref[0, q] + part_ref[1, q] for q in range(4)]

    def plane(p):
        return jnp.concatenate(
            [psum[q][:, 32 * p : 32 * (p + 1)] for q in range(4)], axis=1
        )

    def species_tp(t, wT_ref):
        acc = jnp.zeros(t.shape, jnp.float32)
        for v in range(S):
            acc = acc + a[:, v : v + 1] * jnp.dot(
                t, wT_ref[v], preferred_element_type=jnp.float32
            )
        return acc * c2

    t0 = jnp.dot(plane(0), wl0_ref[...], preferred_element_type=jnp.float32) * c1
    o0 = species_tp(t0, wm0T_ref)
    f0_ref[...] = jnp.dot(o0, wo0_ref[...], preferred_element_type=jnp.float32) * c3
    for p, out_ref in ((1, fx_ref), (2, fy_ref), (3, fz_ref)):
        t = jnp.dot(plane(p), wl1_ref[...], preferred_element_type=jnp.float32) * c1
        o = species_tp(t, wm1T_ref)
        out_ref[...] = jnp.dot(o, wo1_ref[...], preferred_element_type=jnp.float32) * c3


def _node_post(parts, node_attrs, W_lin0f, W_lin1f, W_msg0T, W_msg1T, W_out0,
               W_out1, *, interpret=False):
    N = node_attrs.shape[0]
    NB = 1000
    return pl.pallas_call(
        _node_post_body,
        grid=(N // NB,),
        in_specs=[
            pl.BlockSpec((2, 4, NB, F), lambda i: (0, 0, i, 0)),
            pl.BlockSpec((NB, S), lambda i: (i, 0)),
            pl.BlockSpec((F, F), lambda i: (0, 0)),
            pl.BlockSpec((F, F), lambda i: (0, 0)),
            pl.BlockSpec((S, F, F), lambda i: (0, 0, 0)),
            pl.BlockSpec((S, F, F), lambda i: (0, 0, 0)),
            pl.BlockSpec((F, F), lambda i: (0, 0)),
            pl.BlockSpec((F, F), lambda i: (0, 0)),
        ],
        out_specs=[pl.BlockSpec((NB, F), lambda i: (i, 0)) for _ in range(4)],
        out_shape=[jax.ShapeDtypeStruct((N, F), jnp.float32) for _ in range(4)],
        interpret=interpret,
    )(parts, node_attrs, W_lin0f, W_lin1f, W_msg0T, W_msg1T, W_out0, W_out1)


def _sc_fused(h0q, cw0q, cw1q, edge_attrs, snd, rcv, zeros_np):
    E = snd.shape[0]
    Np = zeros_np.shape[0]
    E2 = E // 2
    ept = E2 // 16
    CH = 80
    NCH = ept // CH
    npt = Np // 16
    mesh = plsc.VectorSubcoreMesh(core_axis_name="c", subcore_axis_name="s")

    @functools.partial(
        pl.kernel,
        out_type=jax.ShapeDtypeStruct((2, 4, Np, F), jnp.float32),
        mesh=mesh,
        scratch_types=[
            pltpu.VMEM_SHARED((Np, F), jnp.float32),
            pltpu.VMEM((CH,), jnp.int32),
            pltpu.VMEM((CH,), jnp.int32),
            pltpu.VMEM((CH, F), jnp.float32),
            pltpu.VMEM((CH, 32), jnp.float32),
            pltpu.VMEM((CH, 32), jnp.float32),
            pltpu.VMEM((CH * 4 + 16,), jnp.float32),
            pltpu.VMEM((CH, F), jnp.float32),
            pltpu.SemaphoreType.DMA,
        ],
    )
    def fk(h0q_hbm, cw0q_hbm, cw1q_hbm, ea_hbm, snd_hbm, rcv_hbm, zero_hbm,
           out_hbm, acc_sh, is_v, ir_v, s0_v, c0_v, c1_v, ea_v, upd_v, sem):
        c = lax.axis_index("c")
        s = lax.axis_index("s")
        ebase = c * E2 + s * ept
        r0 = s * npt
        for q in range(4):
            pltpu.sync_copy(zero_hbm.at[pl.ds(r0, npt)], acc_sh.at[pl.ds(r0, npt)])
            plsc.subcore_barrier()

            def chunk_body(i, carry):
                off = ebase + i * CH
                pltpu.sync_copy(snd_hbm.at[pl.ds(off, CH)], is_v)
                pltpu.sync_copy(rcv_hbm.at[pl.ds(off, CH)], ir_v)
                pltpu.async_copy(h0q_hbm.at[is_v], s0_v, sem).wait()
                pltpu.sync_copy(cw0q_hbm.at[q].at[pl.ds(off, CH)], c0_v)
                pltpu.sync_copy(cw1q_hbm.at[q].at[pl.ds(off, CH)], c1_v)
                pltpu.sync_copy(ea_hbm.at[pl.ds(off * 4, CH * 4)],
                                ea_v.at[pl.ds(0, CH * 4)])

                def edge_body(e, carry2):
                    ywin = ea_v[pl.ds(e * 4, 16)]
                    yx = jnp.full((16,), ywin[1], jnp.float32)
                    yy = jnp.full((16,), ywin[2], jnp.float32)
                    yz = jnp.full((16,), ywin[3], jnp.float32)
                    for h in range(2):
                        sl = pl.ds(16 * h, 16)
                        sv = s0_v[e, pl.ds(32 * q + 16 * h, 16)]
                        b = c1_v[e, sl] * sv
                        upd_v[e, pl.ds(16 * h, 16)] = c0_v[e, sl] * sv
                        upd_v[e, pl.ds(32 + 16 * h, 16)] = b * yx
                        upd_v[e, pl.ds(64 + 16 * h, 16)] = b * yy
                        upd_v[e, pl.ds(96 + 16 * h, 16)] = b * yz
                    return carry2

                lax.fori_loop(0, CH, edge_body, 0)
                pltpu.sync_copy(upd_v, acc_sh.at[ir_v], add=True)
                return carry

            lax.fori_loop(0, NCH, chunk_body, 0)
            plsc.subcore_barrier()
            pltpu.sync_copy(
                acc_sh.at[pl.ds(r0, npt)],
                out_hbm.at[c].at[q].at[pl.ds(r0, npt)],
            )
            plsc.subcore_barrier()

    return fk(h0q, cw0q, cw1q, edge_attrs, snd, rcv, zeros_np)


def kernel(node_attrs, node_feats, edge_attrs, edge_feats, W_skip, W_up, W_r1,
           W_r2, W_r3, W_r4, W_lin0, W_lin1, W_msg0, W_msg1, W_out0, W_out1,
           edge_index, species):
    N = node_feats.shape[0]
    snd = edge_index[0]
    rcv = edge_index[1]

    h0q, skip0 = _node_pre(node_feats, node_attrs, W_up, W_skip.transpose(1, 0, 2))
    cw0q, cw1q = _edge_pre(edge_feats, edge_attrs, W_r1, W_r2, W_r3, W_r4[:, : 2 * F])
    Np = ((N // 16 + 7) // 8 * 8) * 16
    parts = _sc_fused(h0q, cw0q, cw1q, edge_attrs.reshape(-1), snd, rcv,
                      jnp.zeros((Np, F), jnp.float32))
    f0, fx, fy, fz = _node_post(
        parts[:, :, :N], node_attrs, W_lin0[:F], W_lin1[:F],
        W_msg0.transpose(1, 0, 2), W_msg1.transpose(1, 0, 2), W_out0, W_out1,
    )
    message = jnp.concatenate(
        [f0, jnp.stack([fx, fy, fz], axis=-1).reshape(N, 3 * F)], axis=1
    )
    skip = jnp.concatenate([skip0, jnp.zeros((N, 3 * F), jnp.float32)], axis=1)
    return message, skip


# speedup vs baseline: 1.4769x; 1.4769x over previous
import functools

import jax
import jax.numpy as jnp
from jax import lax
from jax.experimental import pallas as pl
from jax.experimental.pallas import tpu as pltpu
from jax.experimental.pallas import tpu_sc as plsc

F = 128
S = 10


def _silu(x):
    return x * (1.0 / (1.0 + jnp.exp(-x)))


def _node_pre_body(nf_ref, na_ref, wup_ref, wskipT_ref, h0q_ref, skip0_ref):
    x = nf_ref[...]
    h0q_ref[...] = jnp.dot(x, wup_ref[...], preferred_element_type=jnp.float32) * (
        F ** -0.5
    )
    a = na_ref[...]
    acc = jnp.zeros(x.shape, jnp.float32)
    for v in range(S):
        acc = acc + a[:, v : v + 1] * jnp.dot(
            x, wskipT_ref[v], preferred_element_type=jnp.float32
        )
    skip0_ref[...] = acc * ((F * S) ** -0.5)


def _node_pre(node_feats, node_attrs, W_up, W_skipT, *, interpret=False):
    N = node_feats.shape[0]
    NB = 1000
    return pl.pallas_call(
        _node_pre_body,
        grid=(N // NB,),
        in_specs=[
            pl.BlockSpec((NB, F), lambda i: (i, 0)),
            pl.BlockSpec((NB, S), lambda i: (i, 0)),
            pl.BlockSpec((F, F), lambda i: (0, 0)),
            pl.BlockSpec((S, F, F), lambda i: (0, 0, 0)),
        ],
        out_specs=[
            pl.BlockSpec((NB, F), lambda i: (i, 0)),
            pl.BlockSpec((NB, F), lambda i: (i, 0)),
        ],
        out_shape=[
            jax.ShapeDtypeStruct((N, F), jnp.float32),
            jax.ShapeDtypeStruct((N, F), jnp.float32),
        ],
        interpret=interpret,
    )(node_feats, node_attrs, W_up, W_skipT)


def _edge_pre_body(ef_ref, ea_ref, wr1_ref, wr2_ref, wr3_ref, wr4_ref,
                   cw0_ref, cw1_ref):
    x = ef_ref[...]
    w = _silu(jnp.dot(x, wr1_ref[...], preferred_element_type=jnp.float32) * (8.0 ** -0.5))
    w = _silu(jnp.dot(w, wr2_ref[...], preferred_element_type=jnp.float32) * (64.0 ** -0.5))
    w = _silu(jnp.dot(w, wr3_ref[...], preferred_element_type=jnp.float32) * (64.0 ** -0.5))
    w = jnp.dot(w, wr4_ref[...], preferred_element_type=jnp.float32) * (64.0 ** -0.5)
    y0 = ea_ref[...][:, 0:1]
    for q in range(4):
        cw0_ref[q] = w[:, 32 * q : 32 * (q + 1)] * y0
        cw1_ref[q] = w[:, F + 32 * q : F + 32 * (q + 1)]


def _edge_pre(edge_feats, edge_attrs, W_r1, W_r2, W_r3, W_r4b, *, interpret=False):
    E = edge_feats.shape[0]
    EB = 2000
    return pl.pallas_call(
        _edge_pre_body,
        grid=(E // EB,),
        in_specs=[
            pl.BlockSpec((EB, 8), lambda i: (i, 0)),
            pl.BlockSpec((EB, 4), lambda i: (i, 0)),
            pl.BlockSpec((8, 64), lambda i: (0, 0)),
            pl.BlockSpec((64, 64), lambda i: (0, 0)),
            pl.BlockSpec((64, 64), lambda i: (0, 0)),
            pl.BlockSpec((64, 2 * F), lambda i: (0, 0)),
        ],
        out_specs=[
            pl.BlockSpec((4, EB, 32), lambda i: (0, i, 0)),
            pl.BlockSpec((4, EB, 32), lambda i: (0, i, 0)),
        ],
        out_shape=[
            jax.ShapeDtypeStruct((4, E, 32), jnp.float32),
            jax.ShapeDtypeStruct((4, E, 32), jnp.float32),
        ],
        interpret=interpret,
    )(edge_feats, edge_attrs, W_r1, W_r2, W_r3, W_r4b)


def _node_post_body(
    part_ref, na_ref, wl0_ref, wl1_ref, wm0T_ref, wm1T_ref, wo0_ref, wo1_ref,
    f0_ref, fx_ref, fy_ref, fz_ref,
):
    c1 = ((2 * F) ** -0.5) * 0.25
    c2 = (F * S) ** -0.5
    c3 = F ** -0.5
    a = na_ref[...]
    psum = [part_ref[0, q] + part_ref[1, q] for q in range(4)]

    def plane(p):
        return jnp.concatenate(
            [psum[q][:, 32 * p : 32 * (p + 1)] for q in range(4)], axis=1
        )

    def species_tp(t, wT_ref):
        acc = jnp.zeros(t.shape, jnp.float32)
        for v in range(S):
            acc = acc + a[:, v : v + 1] * jnp.dot(
                t, wT_ref[v], preferred_element_type=jnp.float32
            )
        return acc * c2

    t0 = jnp.dot(plane(0), wl0_ref[...], preferred_element_type=jnp.float32) * c1
    o0 = species_tp(t0, wm0T_ref)
    f0_ref[...] = jnp.dot(o0, wo0_ref[...], preferred_element_type=jnp.float32) * c3
    for p, out_ref in ((1, fx_ref), (2, fy_ref), (3, fz_ref)):
        t = jnp.dot(plane(p), wl1_ref[...], preferred_element_type=jnp.float32) * c1
        o = species_tp(t, wm1T_ref)
        out_ref[...] = jnp.dot(o, wo1_ref[...], preferred_element_type=jnp.float32) * c3


def _node_post(parts, node_attrs, W_lin0f, W_lin1f, W_msg0T, W_msg1T, W_out0,
               W_out1, *, interpret=False):
    N = node_attrs.shape[0]
    NB = 1000
    return pl.pallas_call(
        _node_post_body,
        grid=(N // NB,),
        in_specs=[
            pl.BlockSpec((2, 4, NB, F), lambda i: (0, 0, i, 0)),
            pl.BlockSpec((NB, S), lambda i: (i, 0)),
            pl.BlockSpec((F, F), lambda i: (0, 0)),
            pl.BlockSpec((F, F), lambda i: (0, 0)),
            pl.BlockSpec((S, F, F), lambda i: (0, 0, 0)),
            pl.BlockSpec((S, F, F), lambda i: (0, 0, 0)),
            pl.BlockSpec((F, F), lambda i: (0, 0)),
            pl.BlockSpec((F, F), lambda i: (0, 0)),
        ],
        out_specs=[pl.BlockSpec((NB, F), lambda i: (i, 0)) for _ in range(4)],
        out_shape=[jax.ShapeDtypeStruct((N, F), jnp.float32) for _ in range(4)],
        interpret=interpret,
    )(parts, node_attrs, W_lin0f, W_lin1f, W_msg0T, W_msg1T, W_out0, W_out1)


def _sc_fused(h0q, cw0q, cw1q, edge_attrs, snd, rcv, zeros_np):
    E = snd.shape[0]
    Np = zeros_np.shape[0]
    E2 = E // 2
    ept = E2 // 16
    CH = 40
    NCH = ept // CH
    npt = Np // 16
    mesh = plsc.VectorSubcoreMesh(core_axis_name="c", subcore_axis_name="s")

    @functools.partial(
        pl.kernel,
        out_type=jax.ShapeDtypeStruct((2, 4, Np, F), jnp.float32),
        mesh=mesh,
        scratch_types=[
            pltpu.VMEM_SHARED((Np, F), jnp.float32),
            pltpu.VMEM((CH,), jnp.int32),
            pltpu.VMEM((CH,), jnp.int32),
            pltpu.VMEM((CH,), jnp.int32),
            pltpu.VMEM((CH,), jnp.int32),
            pltpu.VMEM((2, CH, F), jnp.float32),
            pltpu.VMEM((2, CH, 32), jnp.float32),
            pltpu.VMEM((2, CH, 32), jnp.float32),
            pltpu.VMEM((384,), jnp.float32),
            pltpu.VMEM((384,), jnp.float32),
            pltpu.VMEM((CH, F), jnp.float32),
            pltpu.SemaphoreType.DMA,
            pltpu.SemaphoreType.DMA,
            pltpu.SemaphoreType.DMA,
            pltpu.SemaphoreType.DMA,
        ],
    )
    def fk(h0q_hbm, cw0q_hbm, cw1q_hbm, ea_hbm, snd_hbm, rcv_hbm, zero_hbm,
           out_hbm, acc_sh, is_v0, is_v1, ir_v0, ir_v1, s0_v, c0_v, c1_v,
           ea_v0, ea_v1, upd_v, sem0, sem1, gsem0, gsem1):
        c = lax.axis_index("c")
        s = lax.axis_index("s")
        ebase = c * E2 + s * ept
        r0 = s * npt
        sems = (sem0, sem1)
        gsems = (gsem0, gsem1)
        is_bufs = (is_v0, is_v1)
        ir_bufs = (ir_v0, ir_v1)
        ea_bufs = (ea_v0, ea_v1)

        for q in range(4):

            def lin_pairs(i, b):
                off = ebase + i * CH
                return (
                    (snd_hbm.at[pl.ds(off, CH)], is_bufs[b]),
                    (rcv_hbm.at[pl.ds(off, CH)], ir_bufs[b]),
                    (cw0q_hbm.at[q].at[pl.ds(off, CH)], c0_v.at[b]),
                    (cw1q_hbm.at[q].at[pl.ds(off, CH)], c1_v.at[b]),
                    (ea_hbm.at[pl.ds(off * 4, 384)], ea_bufs[b]),
                )

            def lin_issue(i, b):
                for sr, dr in lin_pairs(i, b):
                    pltpu.async_copy(sr, dr, sems[b])

            def lin_wait(i, b):
                for sr, dr in lin_pairs(i, b):
                    pltpu.make_async_copy(sr, dr, sems[b]).wait()

            def gath_issue(b):
                pltpu.async_copy(h0q_hbm.at[is_bufs[b]], s0_v.at[b], gsems[b])

            def gath_wait(b):
                pltpu.make_async_copy(
                    h0q_hbm.at[is_bufs[b]], s0_v.at[b], gsems[b]
                ).wait()

            def compute_scatter(b):
                @plsc.parallel_loop(0, CH, 1, unroll=4)
                def edge_body(e):
                    ywin = ea_bufs[b][pl.ds(e * 4, 16)]
                    yx = jnp.full((16,), ywin[1], jnp.float32)
                    yy = jnp.full((16,), ywin[2], jnp.float32)
                    yz = jnp.full((16,), ywin[3], jnp.float32)
                    for h in range(2):
                        sl = pl.ds(16 * h, 16)
                        sv = s0_v[b, e, pl.ds(32 * q + 16 * h, 16)]
                        bb = c1_v[b, e, sl] * sv
                        upd_v[e, pl.ds(16 * h, 16)] = c0_v[b, e, sl] * sv
                        upd_v[e, pl.ds(32 + 16 * h, 16)] = bb * yx
                        upd_v[e, pl.ds(64 + 16 * h, 16)] = bb * yy
                        upd_v[e, pl.ds(96 + 16 * h, 16)] = bb * yz

                pltpu.sync_copy(upd_v, acc_sh.at[ir_bufs[b]], add=True)

            pltpu.sync_copy(zero_hbm.at[pl.ds(r0, npt)], acc_sh.at[pl.ds(r0, npt)])
            plsc.subcore_barrier()

            lin_issue(0, 0)
            lin_wait(0, 0)
            gath_issue(0)
            lin_issue(1, 1)

            def pair_body(i2, carry):
                for bi in range(2):
                    i = i2 * 2 + bi
                    gath_wait(bi)
                    lin_wait(i + 1, 1 - bi)
                    gath_issue(1 - bi)
                    compute_scatter(bi)

                    @pl.when(i + 2 < NCH)
                    def _():
                        lin_issue(i + 2, bi)

                return carry

            P2 = 2 * ((NCH - 1) // 2)
            lax.fori_loop(0, (NCH - 1) // 2, pair_body, 0)
            for idx in range(P2, NCH - 1):
                b = idx % 2
                gath_wait(b)
                lin_wait(idx + 1, 1 - b)
                gath_issue(1 - b)
                compute_scatter(b)
                if idx + 2 < NCH:
                    lin_issue(idx + 2, b)
            gath_wait((NCH - 1) % 2)
            compute_scatter((NCH - 1) % 2)

            plsc.subcore_barrier()
            pltpu.sync_copy(
                acc_sh.at[pl.ds(r0, npt)],
                out_hbm.at[c].at[q].at[pl.ds(r0, npt)],
            )
            plsc.subcore_barrier()

    return fk(h0q, cw0q, cw1q, edge_attrs, snd, rcv, zeros_np)


def kernel(node_attrs, node_feats, edge_attrs, edge_feats, W_skip, W_up, W_r1,
           W_r2, W_r3, W_r4, W_lin0, W_lin1, W_msg0, W_msg1, W_out0, W_out1,
           edge_index, species):
    N = node_feats.shape[0]
    snd = edge_index[0]
    rcv = edge_index[1]

    h0q, skip0 = _node_pre(node_feats, node_attrs, W_up, W_skip.transpose(1, 0, 2))
    cw0q, cw1q = _edge_pre(edge_feats, edge_attrs, W_r1, W_r2, W_r3, W_r4[:, : 2 * F])
    Np = ((N // 16 + 7) // 8 * 8) * 16
    ea_flat = jnp.concatenate(
        [edge_attrs.reshape(-1), jnp.zeros((64,), jnp.float32)]
    )
    parts = _sc_fused(h0q, cw0q, cw1q, ea_flat, snd, rcv,
                      jnp.zeros((Np, F), jnp.float32))
    f0, fx, fy, fz = _node_post(
        parts[:, :, :N], node_attrs, W_lin0[:F], W_lin1[:F],
        W_msg0.transpose(1, 0, 2), W_msg1.transpose(1, 0, 2), W_out0, W_out1,
    )
    message = jnp.concatenate(
        [f0, jnp.stack([fx, fy, fz], axis=-1).reshape(N, 3 * F)], axis=1
    )
    skip = jnp.concatenate([skip0, jnp.zeros((N, 3 * F), jnp.float32)], axis=1)
    return message, skip


# speedup vs baseline: 3.0335x; 2.0540x over previous
import functools

import jax
import jax.numpy as jnp
from jax import lax
from jax.experimental import pallas as pl
from jax.experimental.pallas import tpu as pltpu
from jax.experimental.pallas import tpu_sc as plsc

F = 128
S = 10


def _silu(x):
    return x * (1.0 / (1.0 + jnp.exp(-x)))


def _node_pre_body(nf_ref, na_ref, wup_ref, wskipT_ref, h0_ref, skip0_ref):
    x = nf_ref[...]
    h0_ref[...] = jnp.dot(x, wup_ref[...], preferred_element_type=jnp.float32) * (
        F ** -0.5
    )
    a = na_ref[...]
    acc = jnp.zeros(x.shape, jnp.float32)
    for v in range(S):
        acc = acc + a[:, v : v + 1] * jnp.dot(
            x, wskipT_ref[v], preferred_element_type=jnp.float32
        )
    skip0_ref[...] = acc * ((F * S) ** -0.5)


def _node_pre(node_feats, node_attrs, W_up, W_skipT, *, interpret=False):
    N = node_feats.shape[0]
    NB = 1000
    return pl.pallas_call(
        _node_pre_body,
        grid=(N // NB,),
        in_specs=[
            pl.BlockSpec((NB, F), lambda i: (i, 0)),
            pl.BlockSpec((NB, S), lambda i: (i, 0)),
            pl.BlockSpec((F, F), lambda i: (0, 0)),
            pl.BlockSpec((S, F, F), lambda i: (0, 0, 0)),
        ],
        out_specs=[
            pl.BlockSpec((NB, F), lambda i: (i, 0)),
            pl.BlockSpec((NB, F), lambda i: (i, 0)),
        ],
        out_shape=[
            jax.ShapeDtypeStruct((N, F), jnp.float32),
            jax.ShapeDtypeStruct((N, F), jnp.float32),
        ],
        interpret=interpret,
    )(node_feats, node_attrs, W_up, W_skipT)


def _edge_pre_body(ef_ref, ea_ref, s0_ref, wr1_ref, wr2_ref, wr3_ref, wr4_ref, m4_ref):
    x = ef_ref[...]
    w = _silu(jnp.dot(x, wr1_ref[...], preferred_element_type=jnp.float32) * (8.0 ** -0.5))
    w = _silu(jnp.dot(w, wr2_ref[...], preferred_element_type=jnp.float32) * (64.0 ** -0.5))
    w = _silu(jnp.dot(w, wr3_ref[...], preferred_element_type=jnp.float32) * (64.0 ** -0.5))
    w = jnp.dot(w, wr4_ref[...], preferred_element_type=jnp.float32) * (64.0 ** -0.5)
    s = s0_ref[...]
    y = ea_ref[...]
    m4_ref[0] = w[:, :F] * s * y[:, 0:1]
    b = w[:, F:] * s
    m4_ref[1] = b * y[:, 1:2]
    m4_ref[2] = b * y[:, 2:3]
    m4_ref[3] = b * y[:, 3:4]


def _edge_pre(edge_feats, edge_attrs, s0, W_r1, W_r2, W_r3, W_r4b, *, interpret=False):
    E = edge_feats.shape[0]
    EB = 2000
    return pl.pallas_call(
        _edge_pre_body,
        grid=(E // EB,),
        in_specs=[
            pl.BlockSpec((EB, 8), lambda i: (i, 0)),
            pl.BlockSpec((EB, 4), lambda i: (i, 0)),
            pl.BlockSpec((EB, F), lambda i: (i, 0)),
            pl.BlockSpec((8, 64), lambda i: (0, 0)),
            pl.BlockSpec((64, 64), lambda i: (0, 0)),
            pl.BlockSpec((64, 64), lambda i: (0, 0)),
            pl.BlockSpec((64, 2 * F), lambda i: (0, 0)),
        ],
        out_specs=pl.BlockSpec((4, EB, F), lambda i: (0, i, 0)),
        out_shape=jax.ShapeDtypeStruct((4, E, F), jnp.float32),
        interpret=interpret,
    )(edge_feats, edge_attrs, s0, W_r1, W_r2, W_r3, W_r4b)


def _node_post_body(
    msg4_ref, na_ref, wl0_ref, wl1_ref, wm0T_ref, wm1T_ref, wo0_ref, wo1_ref,
    f0_ref, fx_ref, fy_ref, fz_ref,
):
    c1 = ((2 * F) ** -0.5) * 0.25
    c2 = (F * S) ** -0.5
    c3 = F ** -0.5
    a = na_ref[...]

    def species_tp(t, wT_ref):
        acc = jnp.zeros(t.shape, jnp.float32)
        for v in range(S):
            acc = acc + a[:, v : v + 1] * jnp.dot(
                t, wT_ref[v], preferred_element_type=jnp.float32
            )
        return acc * c2

    t0 = jnp.dot(msg4_ref[0], wl0_ref[...], preferred_element_type=jnp.float32) * c1
    o0 = species_tp(t0, wm0T_ref)
    f0_ref[...] = jnp.dot(o0, wo0_ref[...], preferred_element_type=jnp.float32) * c3
    for i, out_ref in ((1, fx_ref), (2, fy_ref), (3, fz_ref)):
        t = jnp.dot(msg4_ref[i], wl1_ref[...], preferred_element_type=jnp.float32) * c1
        o = species_tp(t, wm1T_ref)
        out_ref[...] = jnp.dot(o, wo1_ref[...], preferred_element_type=jnp.float32) * c3


def _node_post(msg4, node_attrs, W_lin0f, W_lin1f, W_msg0T, W_msg1T, W_out0, W_out1,
               *, interpret=False):
    N = node_attrs.shape[0]
    NB = 1000
    return pl.pallas_call(
        _node_post_body,
        grid=(N // NB,),
        in_specs=[
            pl.BlockSpec((4, NB, F), lambda i: (0, i, 0)),
            pl.BlockSpec((NB, S), lambda i: (i, 0)),
            pl.BlockSpec((F, F), lambda i: (0, 0)),
            pl.BlockSpec((F, F), lambda i: (0, 0)),
            pl.BlockSpec((S, F, F), lambda i: (0, 0, 0)),
            pl.BlockSpec((S, F, F), lambda i: (0, 0, 0)),
            pl.BlockSpec((F, F), lambda i: (0, 0)),
            pl.BlockSpec((F, F), lambda i: (0, 0)),
        ],
        out_specs=[pl.BlockSpec((NB, F), lambda i: (i, 0)) for _ in range(4)],
        out_shape=[jax.ShapeDtypeStruct((N, F), jnp.float32) for _ in range(4)],
        interpret=interpret,
    )(msg4, node_attrs, W_lin0f, W_lin1f, W_msg0T, W_msg1T, W_out0, W_out1)


def _sc_gather(h0, snd):
    N = h0.shape[0]
    E = snd.shape[0]
    NW = 32
    e_per_w = E // NW
    CH = 400
    mesh = plsc.VectorSubcoreMesh(core_axis_name="c", subcore_axis_name="s")

    NCH = e_per_w // CH

    @functools.partial(
        pl.kernel,
        out_type=jax.ShapeDtypeStruct((E, F), jnp.float32),
        mesh=mesh,
        scratch_types=[
            pltpu.VMEM((CH,), jnp.int32),
            pltpu.VMEM((CH,), jnp.int32),
            pltpu.VMEM((2, CH, F), jnp.float32),
            pltpu.SemaphoreType.DMA,
            pltpu.SemaphoreType.DMA,
            pltpu.SemaphoreType.DMA,
            pltpu.SemaphoreType.DMA,
        ],
    )
    def gk(h0_hbm, snd_hbm, out_hbm, i_v0, i_v1, rows_v, is0, is1, gs0, gs1):
        wid = lax.axis_index("s") * 2 + lax.axis_index("c")
        base = wid * e_per_w
        idxb = (i_v0, i_v1)
        isems = (is0, is1)
        gsems = (gs0, gs1)

        def idx_issue(i, b):
            pltpu.async_copy(snd_hbm.at[pl.ds(base + i * CH, CH)], idxb[b], isems[b])

        def idx_wait(i, b):
            pltpu.make_async_copy(
                snd_hbm.at[pl.ds(base + i * CH, CH)], idxb[b], isems[b]
            ).wait()

        def g_issue(b):
            pltpu.async_copy(h0_hbm.at[idxb[b]], rows_v.at[b], gsems[b])

        def g_wait(b):
            pltpu.make_async_copy(h0_hbm.at[idxb[b]], rows_v.at[b], gsems[b]).wait()

        def step(i, bi):
            idx_wait(i + 1, 1 - bi)
            g_issue(1 - bi)
            g_wait(bi)
            pltpu.sync_copy(rows_v.at[bi], out_hbm.at[pl.ds(base + i * CH, CH)])

        idx_issue(0, 0)
        idx_wait(0, 0)
        g_issue(0)
        idx_issue(1, 1)

        def pair_body(i2, carry):
            for bi in range(2):
                i = i2 * 2 + bi
                step(i, bi)

                @pl.when(i + 2 < NCH)
                def _():
                    idx_issue(i + 2, bi)

            return carry

        P2 = 2 * ((NCH - 1) // 2)
        lax.fori_loop(0, (NCH - 1) // 2, pair_body, 0)
        for i in range(P2, NCH - 1):
            step(i, i % 2)
            if i + 2 < NCH:
                idx_issue(i + 2, i % 2)
        bl = (NCH - 1) % 2
        g_wait(bl)
        pltpu.sync_copy(
            rows_v.at[bl], out_hbm.at[pl.ds(base + (NCH - 1) * CH, CH)]
        )

    return gk(h0, snd)


def _sc_scatter(m4, rcv, zeros_nf):
    E = rcv.shape[0]
    Np = zeros_nf.shape[0]
    CH = 80
    n_per_tile = Np // 16
    e_per_tile = E // 16
    mesh = plsc.VectorSubcoreMesh(core_axis_name="c", subcore_axis_name="s")

    NCH = e_per_tile // CH

    @functools.partial(
        pl.kernel,
        out_type=jax.ShapeDtypeStruct((4, Np, F), jnp.float32),
        mesh=mesh,
        scratch_types=[
            pltpu.VMEM_SHARED((Np, F), jnp.float32),
            pltpu.VMEM((CH,), jnp.int32),
            pltpu.VMEM((CH,), jnp.int32),
            pltpu.VMEM((2, CH, F), jnp.float32),
            pltpu.SemaphoreType.DMA,
            pltpu.SemaphoreType.DMA,
        ],
    )
    def sk(m4_hbm, rcv_hbm, zero_hbm, out_hbm, acc_sh, i_v0, i_v1, m_v,
           ls0, ls1):
        c = lax.axis_index("c")
        s = lax.axis_index("s")
        r0 = s * n_per_tile
        idxb = (i_v0, i_v1)
        lsems = (ls0, ls1)
        for j in range(2):
            p = c * 2 + j

            def ld_pairs(i, b):
                off = s * e_per_tile + i * CH
                return (
                    (rcv_hbm.at[pl.ds(off, CH)], idxb[b]),
                    (m4_hbm.at[p].at[pl.ds(off, CH)], m_v.at[b]),
                )

            def ld_issue(i, b):
                for sr, dr in ld_pairs(i, b):
                    pltpu.async_copy(sr, dr, lsems[b])

            def ld_wait(i, b):
                for sr, dr in ld_pairs(i, b):
                    pltpu.make_async_copy(sr, dr, lsems[b]).wait()

            def step(i, bi):
                ld_wait(i, bi)
                pltpu.sync_copy(m_v.at[bi], acc_sh.at[idxb[bi]], add=True)

            pltpu.sync_copy(
                zero_hbm.at[pl.ds(r0, n_per_tile)],
                acc_sh.at[pl.ds(r0, n_per_tile)],
            )
            plsc.subcore_barrier()

            ld_issue(0, 0)
            ld_issue(1, 1)

            def pair_body(i2, carry):
                for bi in range(2):
                    i = i2 * 2 + bi
                    step(i, bi)

                    @pl.when(i + 2 < NCH)
                    def _():
                        ld_issue(i + 2, bi)

                return carry

            P2 = 2 * (NCH // 2)
            lax.fori_loop(0, NCH // 2, pair_body, 0)
            for i in range(P2, NCH):
                step(i, i % 2)
            plsc.subcore_barrier()
            pltpu.sync_copy(
                acc_sh.at[pl.ds(r0, n_per_tile)],
                out_hbm.at[p].at[pl.ds(r0, n_per_tile)],
            )
            plsc.subcore_barrier()

    return sk(m4, rcv, zeros_nf)


def kernel(node_attrs, node_feats, edge_attrs, edge_feats, W_skip, W_up, W_r1,
           W_r2, W_r3, W_r4, W_lin0, W_lin1, W_msg0, W_msg1, W_out0, W_out1,
           edge_index, species):
    N = node_feats.shape[0]
    snd = edge_index[0]
    rcv = edge_index[1]

    h0, skip0 = _node_pre(node_feats, node_attrs, W_up, W_skip.transpose(1, 0, 2))
    s0 = _sc_gather(h0, snd)
    m4 = _edge_pre(edge_feats, edge_attrs, s0, W_r1, W_r2, W_r3, W_r4[:, : 2 * F])
    Np = ((N // 16 + 7) // 8 * 8) * 16
    msg4 = _sc_scatter(m4, rcv, jnp.zeros((Np, F), jnp.float32))[:, :N]
    f0, fx, fy, fz = _node_post(
        msg4, node_attrs, W_lin0[:F], W_lin1[:F],
        W_msg0.transpose(1, 0, 2), W_msg1.transpose(1, 0, 2), W_out0, W_out1,
    )
    message = jnp.concatenate(
        [f0, jnp.stack([fx, fy, fz], axis=-1).reshape(N, 3 * F)], axis=1
    )
    skip = jnp.concatenate([skip0, jnp.zeros((N, 3 * F), jnp.float32)], axis=1)
    return message, skip


# speedup vs baseline: 3.2588x; 1.0743x over previous
import functools

import jax
import jax.numpy as jnp
from jax import lax
from jax.experimental import pallas as pl
from jax.experimental.pallas import tpu as pltpu
from jax.experimental.pallas import tpu_sc as plsc

F = 128
S = 10


def _silu(x):
    return x * (1.0 / (1.0 + jnp.exp(-x)))


def _node_pre_body(nf_ref, na_ref, wup_ref, wskipT_ref, h0_ref, skip0_ref):
    x = nf_ref[...]
    h0_ref[...] = jnp.dot(x, wup_ref[...], preferred_element_type=jnp.float32) * (
        F ** -0.5
    )
    a = na_ref[...]
    acc = jnp.zeros(x.shape, jnp.float32)
    for v in range(S):
        acc = acc + a[:, v : v + 1] * jnp.dot(
            x, wskipT_ref[v], preferred_element_type=jnp.float32
        )
    skip0_ref[...] = acc * ((F * S) ** -0.5)


def _node_pre(node_feats, node_attrs, W_up, W_skipT, *, interpret=False):
    N = node_feats.shape[0]
    NB = 1000
    return pl.pallas_call(
        _node_pre_body,
        grid=(N // NB,),
        in_specs=[
            pl.BlockSpec((NB, F), lambda i: (i, 0)),
            pl.BlockSpec((NB, S), lambda i: (i, 0)),
            pl.BlockSpec((F, F), lambda i: (0, 0)),
            pl.BlockSpec((S, F, F), lambda i: (0, 0, 0)),
        ],
        out_specs=[
            pl.BlockSpec((NB, F), lambda i: (i, 0)),
            pl.BlockSpec((NB, F), lambda i: (i, 0)),
        ],
        out_shape=[
            jax.ShapeDtypeStruct((N, F), jnp.float32),
            jax.ShapeDtypeStruct((N, F), jnp.float32),
        ],
        interpret=interpret,
    )(node_feats, node_attrs, W_up, W_skipT)


def _edge_pre_body(ef_ref, ea_ref, s0_ref, wr1_ref, wr2_ref, wr3_ref, wr4_ref, m4_ref):
    x = ef_ref[...]
    w = _silu(jnp.dot(x, wr1_ref[...], preferred_element_type=jnp.float32) * (8.0 ** -0.5))
    w = _silu(jnp.dot(w, wr2_ref[...], preferred_element_type=jnp.float32) * (64.0 ** -0.5))
    w = _silu(jnp.dot(w, wr3_ref[...], preferred_element_type=jnp.float32) * (64.0 ** -0.5))
    w = jnp.dot(w, wr4_ref[...], preferred_element_type=jnp.float32) * (64.0 ** -0.5)
    s = s0_ref[...]
    y = ea_ref[...]
    m4_ref[0] = w[:, :F] * s * y[:, 0:1]
    b = w[:, F:] * s
    m4_ref[1] = b * y[:, 1:2]
    m4_ref[2] = b * y[:, 2:3]
    m4_ref[3] = b * y[:, 3:4]


def _edge_pre(edge_feats, edge_attrs, s0, W_r1, W_r2, W_r3, W_r4b, *, interpret=False):
    E = edge_feats.shape[0]
    EB = 2000
    return pl.pallas_call(
        _edge_pre_body,
        grid=(E // EB,),
        in_specs=[
            pl.BlockSpec((EB, 8), lambda i: (i, 0)),
            pl.BlockSpec((EB, 4), lambda i: (i, 0)),
            pl.BlockSpec((EB, F), lambda i: (i, 0)),
            pl.BlockSpec((8, 64), lambda i: (0, 0)),
            pl.BlockSpec((64, 64), lambda i: (0, 0)),
            pl.BlockSpec((64, 64), lambda i: (0, 0)),
            pl.BlockSpec((64, 2 * F), lambda i: (0, 0)),
        ],
        out_specs=pl.BlockSpec((4, EB, F), lambda i: (0, i, 0)),
        out_shape=jax.ShapeDtypeStruct((4, E, F), jnp.float32),
        interpret=interpret,
    )(edge_feats, edge_attrs, s0, W_r1, W_r2, W_r3, W_r4b)


def _node_post_body(
    msg4_ref, na_ref, wl0_ref, wl1_ref, wm0T_ref, wm1T_ref, wo0_ref, wo1_ref,
    f0_ref, fx_ref, fy_ref, fz_ref,
):
    c1 = ((2 * F) ** -0.5) * 0.25
    c2 = (F * S) ** -0.5
    c3 = F ** -0.5
    a = na_ref[...]

    def species_tp(t, wT_ref):
        acc = jnp.zeros(t.shape, jnp.float32)
        for v in range(S):
            acc = acc + a[:, v : v + 1] * jnp.dot(
                t, wT_ref[v], preferred_element_type=jnp.float32
            )
        return acc * c2

    t0 = jnp.dot(msg4_ref[0], wl0_ref[...], preferred_element_type=jnp.float32) * c1
    o0 = species_tp(t0, wm0T_ref)
    f0_ref[...] = jnp.dot(o0, wo0_ref[...], preferred_element_type=jnp.float32) * c3
    for i, out_ref in ((1, fx_ref), (2, fy_ref), (3, fz_ref)):
        t = jnp.dot(msg4_ref[i], wl1_ref[...], preferred_element_type=jnp.float32) * c1
        o = species_tp(t, wm1T_ref)
        out_ref[...] = jnp.dot(o, wo1_ref[...], preferred_element_type=jnp.float32) * c3


def _node_post(msg4, node_attrs, W_lin0f, W_lin1f, W_msg0T, W_msg1T, W_out0, W_out1,
               *, interpret=False):
    N = node_attrs.shape[0]
    NB = 1000
    return pl.pallas_call(
        _node_post_body,
        grid=(N // NB,),
        in_specs=[
            pl.BlockSpec((4, NB, F), lambda i: (0, i, 0)),
            pl.BlockSpec((NB, S), lambda i: (i, 0)),
            pl.BlockSpec((F, F), lambda i: (0, 0)),
            pl.BlockSpec((F, F), lambda i: (0, 0)),
            pl.BlockSpec((S, F, F), lambda i: (0, 0, 0)),
            pl.BlockSpec((S, F, F), lambda i: (0, 0, 0)),
            pl.BlockSpec((F, F), lambda i: (0, 0)),
            pl.BlockSpec((F, F), lambda i: (0, 0)),
        ],
        out_specs=[pl.BlockSpec((NB, F), lambda i: (i, 0)) for _ in range(4)],
        out_shape=[jax.ShapeDtypeStruct((N, F), jnp.float32) for _ in range(4)],
        interpret=interpret,
    )(msg4, node_attrs, W_lin0f, W_lin1f, W_msg0T, W_msg1T, W_out0, W_out1)


def _sc_gather(h0, snd):
    N = h0.shape[0]
    E = snd.shape[0]
    NW = 32
    e_per_w = E // NW
    CH = 200
    mesh = plsc.VectorSubcoreMesh(core_axis_name="c", subcore_axis_name="s")

    NCH = e_per_w // CH

    @functools.partial(
        pl.kernel,
        out_type=jax.ShapeDtypeStruct((E, F), jnp.float32),
        mesh=mesh,
        scratch_types=[
            pltpu.VMEM((CH,), jnp.int32),
            pltpu.VMEM((CH,), jnp.int32),
            pltpu.VMEM((2, CH, F), jnp.float32),
            pltpu.SemaphoreType.DMA,
            pltpu.SemaphoreType.DMA,
            pltpu.SemaphoreType.DMA,
            pltpu.SemaphoreType.DMA,
        ],
    )
    def gk(h0_hbm, snd_hbm, out_hbm, i_v0, i_v1, rows_v, is0, is1, gs0, gs1):
        wid = lax.axis_index("s") * 2 + lax.axis_index("c")
        base = wid * e_per_w
        idxb = (i_v0, i_v1)
        isems = (is0, is1)
        gsems = (gs0, gs1)

        def idx_issue(i, b):
            pltpu.async_copy(snd_hbm.at[pl.ds(base + i * CH, CH)], idxb[b], isems[b])

        def idx_wait(i, b):
            pltpu.make_async_copy(
                snd_hbm.at[pl.ds(base + i * CH, CH)], idxb[b], isems[b]
            ).wait()

        def g_issue(b):
            pltpu.async_copy(h0_hbm.at[idxb[b]], rows_v.at[b], gsems[b])

        def g_wait(b):
            pltpu.make_async_copy(h0_hbm.at[idxb[b]], rows_v.at[b], gsems[b]).wait()

        def step(i, bi):
            idx_wait(i + 1, 1 - bi)
            g_issue(1 - bi)
            g_wait(bi)
            pltpu.sync_copy(rows_v.at[bi], out_hbm.at[pl.ds(base + i * CH, CH)])

        idx_issue(0, 0)
        idx_wait(0, 0)
        g_issue(0)
        idx_issue(1, 1)

        def pair_body(i2, carry):
            for bi in range(2):
                i = i2 * 2 + bi
                step(i, bi)

                @pl.when(i + 2 < NCH)
                def _():
                    idx_issue(i + 2, bi)

            return carry

        P2 = 2 * ((NCH - 1) // 2)
        lax.fori_loop(0, (NCH - 1) // 2, pair_body, 0)
        for i in range(P2, NCH - 1):
            step(i, i % 2)
            if i + 2 < NCH:
                idx_issue(i + 2, i % 2)
        bl = (NCH - 1) % 2
        g_wait(bl)
        pltpu.sync_copy(
            rows_v.at[bl], out_hbm.at[pl.ds(base + (NCH - 1) * CH, CH)]
        )

    return gk(h0, snd)


def _sc_scatter(m4, rcv, init4):
    E = rcv.shape[0]
    Np = init4.shape[1]
    CH = 80
    n_per_tile = Np // 16
    e_per_tile = E // 16
    mesh = plsc.VectorSubcoreMesh(core_axis_name="c", subcore_axis_name="s")

    NCH = e_per_tile // CH

    @functools.partial(
        pl.kernel,
        out_type=jax.ShapeDtypeStruct((4, Np, F), jnp.float32),
        mesh=mesh,
        scratch_types=[
            pltpu.VMEM_SHARED((Np, F), jnp.float32),
            pltpu.VMEM((CH,), jnp.int32),
            pltpu.VMEM((CH,), jnp.int32),
            pltpu.VMEM((2, CH, F), jnp.float32),
            pltpu.SemaphoreType.DMA,
            pltpu.SemaphoreType.DMA,
        ],
    )
    def sk(m4_hbm, rcv_hbm, init4_hbm, out_hbm, acc_sh, i_v0, i_v1, m_v,
           ls0, ls1):
        c = lax.axis_index("c")
        s = lax.axis_index("s")
        r0 = s * n_per_tile
        idxb = (i_v0, i_v1)
        lsems = (ls0, ls1)
        for j in range(2):
            p = c * 2 + j

            def ld_pairs(i, b):
                off = s * e_per_tile + i * CH
                return (
                    (rcv_hbm.at[pl.ds(off, CH)], idxb[b]),
                    (m4_hbm.at[p].at[pl.ds(off, CH)], m_v.at[b]),
                )

            def ld_issue(i, b):
                for sr, dr in ld_pairs(i, b):
                    pltpu.async_copy(sr, dr, lsems[b])

            def ld_wait(i, b):
                for sr, dr in ld_pairs(i, b):
                    pltpu.make_async_copy(sr, dr, lsems[b]).wait()

            def step(i, bi):
                ld_wait(i, bi)
                pltpu.sync_copy(m_v.at[bi], acc_sh.at[idxb[bi]], add=True)

            pltpu.sync_copy(
                init4_hbm.at[p].at[pl.ds(r0, n_per_tile)],
                acc_sh.at[pl.ds(r0, n_per_tile)],
            )
            plsc.subcore_barrier()

            ld_issue(0, 0)
            ld_issue(1, 1)

            def pair_body(i2, carry):
                for bi in range(2):
                    i = i2 * 2 + bi
                    step(i, bi)

                    @pl.when(i + 2 < NCH)
                    def _():
                        ld_issue(i + 2, bi)

                return carry

            P2 = 2 * (NCH // 2)
            lax.fori_loop(0, NCH // 2, pair_body, 0)
            for i in range(P2, NCH):
                step(i, i % 2)
            plsc.subcore_barrier()
            pltpu.sync_copy(
                acc_sh.at[pl.ds(r0, n_per_tile)],
                out_hbm.at[p].at[pl.ds(r0, n_per_tile)],
            )
            plsc.subcore_barrier()

    return sk(m4, rcv, init4)


def kernel(node_attrs, node_feats, edge_attrs, edge_feats, W_skip, W_up, W_r1,
           W_r2, W_r3, W_r4, W_lin0, W_lin1, W_msg0, W_msg1, W_out0, W_out1,
           edge_index, species):
    N = node_feats.shape[0]
    snd = edge_index[0]
    rcv = edge_index[1]

    h0, skip0 = _node_pre(node_feats, node_attrs, W_up, W_skip.transpose(1, 0, 2))
    Np = ((N // 16 + 7) // 8 * 8) * 16
    E = snd.shape[0]
    EH = E // 2
    W_r4b = W_r4[:, : 2 * F]
    parts = jnp.zeros((4, Np, F), jnp.float32)
    for lo in (0, EH):
        s0_h = _sc_gather(h0, lax.slice(snd, (lo,), (lo + EH,)))
        m4_h = _edge_pre(
            lax.slice(edge_feats, (lo, 0), (lo + EH, 8)),
            lax.slice(edge_attrs, (lo, 0), (lo + EH, 4)),
            s0_h, W_r1, W_r2, W_r3, W_r4b,
        )
        parts = _sc_scatter(m4_h, lax.slice(rcv, (lo,), (lo + EH,)), parts)
    msg4 = parts[:, :N]
    f0, fx, fy, fz = _node_post(
        msg4, node_attrs, W_lin0[:F], W_lin1[:F],
        W_msg0.transpose(1, 0, 2), W_msg1.transpose(1, 0, 2), W_out0, W_out1,
    )
    message = jnp.concatenate(
        [f0, jnp.stack([fx, fy, fz], axis=-1).reshape(N, 3 * F)], axis=1
    )
    skip = jnp.concatenate([skip0, jnp.zeros((N, 3 * F), jnp.float32)], axis=1)
    return message, skip


# speedup vs baseline: 3.2811x; 1.0069x over previous
import functools

import jax
import jax.numpy as jnp
from jax import lax
from jax.experimental import pallas as pl
from jax.experimental.pallas import tpu as pltpu
from jax.experimental.pallas import tpu_sc as plsc

F = 128
S = 10


def _silu(x):
    return x * (1.0 / (1.0 + jnp.exp(-x)))


def _node_pre_body(nf_ref, na_ref, wup_ref, wskipT_ref, h0_ref, skip0_ref):
    x = nf_ref[...]
    h0_ref[...] = jnp.dot(x, wup_ref[...], preferred_element_type=jnp.float32) * (
        F ** -0.5
    )
    a = na_ref[...]
    acc = jnp.zeros(x.shape, jnp.float32)
    for v in range(S):
        acc = acc + a[:, v : v + 1] * jnp.dot(
            x, wskipT_ref[v], preferred_element_type=jnp.float32
        )
    skip0_ref[...] = acc * ((F * S) ** -0.5)


def _node_pre(node_feats, node_attrs, W_up, W_skipT, *, interpret=False):
    N = node_feats.shape[0]
    NB = 1000
    return pl.pallas_call(
        _node_pre_body,
        grid=(N // NB,),
        in_specs=[
            pl.BlockSpec((NB, F), lambda i: (i, 0)),
            pl.BlockSpec((NB, S), lambda i: (i, 0)),
            pl.BlockSpec((F, F), lambda i: (0, 0)),
            pl.BlockSpec((S, F, F), lambda i: (0, 0, 0)),
        ],
        out_specs=[
            pl.BlockSpec((NB, F), lambda i: (i, 0)),
            pl.BlockSpec((NB, F), lambda i: (i, 0)),
        ],
        out_shape=[
            jax.ShapeDtypeStruct((N, F), jnp.float32),
            jax.ShapeDtypeStruct((N, F), jnp.float32),
        ],
        interpret=interpret,
    )(node_feats, node_attrs, W_up, W_skipT)


def _edge_pre_body(ef_ref, ea_ref, s0_ref, wr1_ref, wr2_ref, wr3_ref, wr4_ref, m4_ref):
    x = ef_ref[...]
    w = _silu(jnp.dot(x, wr1_ref[...], preferred_element_type=jnp.float32) * (8.0 ** -0.5))
    w = _silu(jnp.dot(w, wr2_ref[...], preferred_element_type=jnp.float32) * (64.0 ** -0.5))
    w = _silu(jnp.dot(w, wr3_ref[...], preferred_element_type=jnp.float32) * (64.0 ** -0.5))
    w = jnp.dot(w, wr4_ref[...], preferred_element_type=jnp.float32) * (64.0 ** -0.5)
    s = s0_ref[...]
    y = ea_ref[...]
    m4_ref[0] = w[:, :F] * s * y[:, 0:1]
    b = w[:, F:] * s
    m4_ref[1] = b * y[:, 1:2]
    m4_ref[2] = b * y[:, 2:3]
    m4_ref[3] = b * y[:, 3:4]


def _edge_pre(edge_feats, edge_attrs, s0, W_r1, W_r2, W_r3, W_r4b, *, interpret=False):
    E = edge_feats.shape[0]
    EB = 2000
    return pl.pallas_call(
        _edge_pre_body,
        grid=(E // EB,),
        in_specs=[
            pl.BlockSpec((EB, 8), lambda i: (i, 0)),
            pl.BlockSpec((EB, 4), lambda i: (i, 0)),
            pl.BlockSpec((EB, F), lambda i: (i, 0)),
            pl.BlockSpec((8, 64), lambda i: (0, 0)),
            pl.BlockSpec((64, 64), lambda i: (0, 0)),
            pl.BlockSpec((64, 64), lambda i: (0, 0)),
            pl.BlockSpec((64, 2 * F), lambda i: (0, 0)),
        ],
        out_specs=pl.BlockSpec((4, EB, F), lambda i: (0, i, 0)),
        out_shape=jax.ShapeDtypeStruct((4, E, F), jnp.float32),
        interpret=interpret,
    )(edge_feats, edge_attrs, s0, W_r1, W_r2, W_r3, W_r4b)


def _node_post_body(
    p1_ref, p2_ref, na_ref, wl0_ref, wl1_ref, wm0T_ref, wm1T_ref, wo0_ref,
    wo1_ref, f0_ref, fx_ref, fy_ref, fz_ref,
):
    c1 = ((2 * F) ** -0.5) * 0.25
    c2 = (F * S) ** -0.5
    c3 = F ** -0.5
    a = na_ref[...]

    def species_tp(t, wT_ref):
        acc = jnp.zeros(t.shape, jnp.float32)
        for v in range(S):
            acc = acc + a[:, v : v + 1] * jnp.dot(
                t, wT_ref[v], preferred_element_type=jnp.float32
            )
        return acc * c2

    m0 = p1_ref[0] + p2_ref[0]
    t0 = jnp.dot(m0, wl0_ref[...], preferred_element_type=jnp.float32) * c1
    o0 = species_tp(t0, wm0T_ref)
    f0_ref[...] = jnp.dot(o0, wo0_ref[...], preferred_element_type=jnp.float32) * c3
    for i, out_ref in ((1, fx_ref), (2, fy_ref), (3, fz_ref)):
        mi = p1_ref[i] + p2_ref[i]
        t = jnp.dot(mi, wl1_ref[...], preferred_element_type=jnp.float32) * c1
        o = species_tp(t, wm1T_ref)
        out_ref[...] = jnp.dot(o, wo1_ref[...], preferred_element_type=jnp.float32) * c3


def _node_post(p1, p2, node_attrs, W_lin0f, W_lin1f, W_msg0T, W_msg1T, W_out0,
               W_out1, *, interpret=False):
    N = node_attrs.shape[0]
    NB = 1000
    return pl.pallas_call(
        _node_post_body,
        grid=(N // NB,),
        in_specs=[
            pl.BlockSpec((4, NB, F), lambda i: (0, i, 0)),
            pl.BlockSpec((4, NB, F), lambda i: (0, i, 0)),
            pl.BlockSpec((NB, S), lambda i: (i, 0)),
            pl.BlockSpec((F, F), lambda i: (0, 0)),
            pl.BlockSpec((F, F), lambda i: (0, 0)),
            pl.BlockSpec((S, F, F), lambda i: (0, 0, 0)),
            pl.BlockSpec((S, F, F), lambda i: (0, 0, 0)),
            pl.BlockSpec((F, F), lambda i: (0, 0)),
            pl.BlockSpec((F, F), lambda i: (0, 0)),
        ],
        out_specs=[pl.BlockSpec((NB, F), lambda i: (i, 0)) for _ in range(4)],
        out_shape=[jax.ShapeDtypeStruct((N, F), jnp.float32) for _ in range(4)],
        interpret=interpret,
    )(p1, p2, node_attrs, W_lin0f, W_lin1f, W_msg0T, W_msg1T, W_out0, W_out1)


def _sc_gather(h0, snd):
    N = h0.shape[0]
    E = snd.shape[0]
    NW = 32
    e_per_w = E // NW
    CH = 200
    mesh = plsc.VectorSubcoreMesh(core_axis_name="c", subcore_axis_name="s")

    NCH = e_per_w // CH

    @functools.partial(
        pl.kernel,
        out_type=jax.ShapeDtypeStruct((E, F), jnp.float32),
        mesh=mesh,
        scratch_types=[
            pltpu.VMEM((CH,), jnp.int32),
            pltpu.VMEM((CH,), jnp.int32),
            pltpu.VMEM((2, CH, F), jnp.float32),
            pltpu.SemaphoreType.DMA,
            pltpu.SemaphoreType.DMA,
            pltpu.SemaphoreType.DMA,
            pltpu.SemaphoreType.DMA,
        ],
    )
    def gk(h0_hbm, snd_hbm, out_hbm, i_v0, i_v1, rows_v, is0, is1, gs0, gs1):
        wid = lax.axis_index("s") * 2 + lax.axis_index("c")
        base = wid * e_per_w
        idxb = (i_v0, i_v1)
        isems = (is0, is1)
        gsems = (gs0, gs1)

        def idx_issue(i, b):
            pltpu.async_copy(snd_hbm.at[pl.ds(base + i * CH, CH)], idxb[b], isems[b])

        def idx_wait(i, b):
            pltpu.make_async_copy(
                snd_hbm.at[pl.ds(base + i * CH, CH)], idxb[b], isems[b]
            ).wait()

        def g_issue(b):
            pltpu.async_copy(h0_hbm.at[idxb[b]], rows_v.at[b], gsems[b])

        def g_wait(b):
            pltpu.make_async_copy(h0_hbm.at[idxb[b]], rows_v.at[b], gsems[b]).wait()

        def step(i, bi):
            idx_wait(i + 1, 1 - bi)
            g_issue(1 - bi)
            g_wait(bi)
            pltpu.sync_copy(rows_v.at[bi], out_hbm.at[pl.ds(base + i * CH, CH)])

        idx_issue(0, 0)
        idx_wait(0, 0)
        g_issue(0)
        idx_issue(1, 1)

        def pair_body(i2, carry):
            for bi in range(2):
                i = i2 * 2 + bi
                step(i, bi)

                @pl.when(i + 2 < NCH)
                def _():
                    idx_issue(i + 2, bi)

            return carry

        P2 = 2 * ((NCH - 1) // 2)
        lax.fori_loop(0, (NCH - 1) // 2, pair_body, 0)
        for i in range(P2, NCH - 1):
            step(i, i % 2)
            if i + 2 < NCH:
                idx_issue(i + 2, i % 2)
        bl = (NCH - 1) % 2
        g_wait(bl)
        pltpu.sync_copy(
            rows_v.at[bl], out_hbm.at[pl.ds(base + (NCH - 1) * CH, CH)]
        )

    return gk(h0, snd)


def _sc_scatter(m4, rcv, init4):
    E = rcv.shape[0]
    Np = init4.shape[1]
    CH = 80
    n_per_tile = Np // 16
    e_per_tile = E // 16
    mesh = plsc.VectorSubcoreMesh(core_axis_name="c", subcore_axis_name="s")

    NCH = e_per_tile // CH

    @functools.partial(
        pl.kernel,
        out_type=jax.ShapeDtypeStruct((4, Np, F), jnp.float32),
        mesh=mesh,
        scratch_types=[
            pltpu.VMEM_SHARED((Np, F), jnp.float32),
            pltpu.VMEM((CH,), jnp.int32),
            pltpu.VMEM((CH,), jnp.int32),
            pltpu.VMEM((2, CH, F), jnp.float32),
            pltpu.SemaphoreType.DMA,
            pltpu.SemaphoreType.DMA,
        ],
    )
    def sk(m4_hbm, rcv_hbm, init4_hbm, out_hbm, acc_sh, i_v0, i_v1, m_v,
           ls0, ls1):
        c = lax.axis_index("c")
        s = lax.axis_index("s")
        r0 = s * n_per_tile
        idxb = (i_v0, i_v1)
        lsems = (ls0, ls1)
        for j in range(2):
            p = c * 2 + j

            def ld_pairs(i, b):
                off = s * e_per_tile + i * CH
                return (
                    (rcv_hbm.at[pl.ds(off, CH)], idxb[b]),
                    (m4_hbm.at[p].at[pl.ds(off, CH)], m_v.at[b]),
                )

            def ld_issue(i, b):
                for sr, dr in ld_pairs(i, b):
                    pltpu.async_copy(sr, dr, lsems[b])

            def ld_wait(i, b):
                for sr, dr in ld_pairs(i, b):
                    pltpu.make_async_copy(sr, dr, lsems[b]).wait()

            def step(i, bi):
                ld_wait(i, bi)
                pltpu.sync_copy(m_v.at[bi], acc_sh.at[idxb[bi]], add=True)

            pltpu.sync_copy(
                init4_hbm.at[p].at[pl.ds(r0, n_per_tile)],
                acc_sh.at[pl.ds(r0, n_per_tile)],
            )
            plsc.subcore_barrier()

            ld_issue(0, 0)
            ld_issue(1, 1)

            def pair_body(i2, carry):
                for bi in range(2):
                    i = i2 * 2 + bi
                    step(i, bi)

                    @pl.when(i + 2 < NCH)
                    def _():
                        ld_issue(i + 2, bi)

                return carry

            P2 = 2 * (NCH // 2)
            lax.fori_loop(0, NCH // 2, pair_body, 0)
            for i in range(P2, NCH):
                step(i, i % 2)
            plsc.subcore_barrier()
            pltpu.sync_copy(
                acc_sh.at[pl.ds(r0, n_per_tile)],
                out_hbm.at[p].at[pl.ds(r0, n_per_tile)],
            )
            plsc.subcore_barrier()

    return sk(m4, rcv, init4)


def kernel(node_attrs, node_feats, edge_attrs, edge_feats, W_skip, W_up, W_r1,
           W_r2, W_r3, W_r4, W_lin0, W_lin1, W_msg0, W_msg1, W_out0, W_out1,
           edge_index, species):
    N = node_feats.shape[0]
    snd = edge_index[0]
    rcv = edge_index[1]

    h0, skip0 = _node_pre(node_feats, node_attrs, W_up, W_skip.transpose(1, 0, 2))
    Np = ((N // 16 + 7) // 8 * 8) * 16
    E = snd.shape[0]
    EH = E // 2
    W_r4b = W_r4[:, : 2 * F]
    zeros4 = jnp.zeros((4, Np, F), jnp.float32)
    parts = []
    for lo in (0, EH):
        s0_h = _sc_gather(h0, lax.slice(snd, (lo,), (lo + EH,)))
        m4_h = _edge_pre(
            lax.slice(edge_feats, (lo, 0), (lo + EH, 8)),
            lax.slice(edge_attrs, (lo, 0), (lo + EH, 4)),
            s0_h, W_r1, W_r2, W_r3, W_r4b,
        )
        parts.append(_sc_scatter(m4_h, lax.slice(rcv, (lo,), (lo + EH,)), zeros4))
    f0, fx, fy, fz = _node_post(
        parts[0], parts[1], node_attrs, W_lin0[:F], W_lin1[:F],
        W_msg0.transpose(1, 0, 2), W_msg1.transpose(1, 0, 2), W_out0, W_out1,
    )
    message = jnp.concatenate(
        [f0, jnp.stack([fx, fy, fz], axis=-1).reshape(N, 3 * F)], axis=1
    )
    skip = jnp.concatenate([skip0, jnp.zeros((N, 3 * F), jnp.float32)], axis=1)
    return message, skip


# speedup vs baseline: 3.2961x; 1.0046x over previous
import functools

import jax
import jax.numpy as jnp
from jax import lax
from jax.experimental import pallas as pl
from jax.experimental.pallas import tpu as pltpu
from jax.experimental.pallas import tpu_sc as plsc

F = 128
S = 10


def _silu(x):
    return x * (1.0 / (1.0 + jnp.exp(-x)))


def _node_pre_body(nf_ref, na_ref, wup_ref, wskipT_ref, h0_ref, skip0_ref):
    x = nf_ref[...]
    h0_ref[...] = jnp.dot(x, wup_ref[...], preferred_element_type=jnp.float32) * (
        F ** -0.5
    )
    a = na_ref[...]
    acc = jnp.zeros(x.shape, jnp.float32)
    for v in range(S):
        acc = acc + a[:, v : v + 1] * jnp.dot(
            x, wskipT_ref[v], preferred_element_type=jnp.float32
        )
    skip0_ref[...] = acc * ((F * S) ** -0.5)


def _node_pre(node_feats, node_attrs, W_up, W_skipT, *, interpret=False):
    N = node_feats.shape[0]
    NB = 1000
    return pl.pallas_call(
        _node_pre_body,
        grid=(N // NB,),
        in_specs=[
            pl.BlockSpec((NB, F), lambda i: (i, 0)),
            pl.BlockSpec((NB, S), lambda i: (i, 0)),
            pl.BlockSpec((F, F), lambda i: (0, 0)),
            pl.BlockSpec((S, F, F), lambda i: (0, 0, 0)),
        ],
        out_specs=[
            pl.BlockSpec((NB, F), lambda i: (i, 0)),
            pl.BlockSpec((NB, F), lambda i: (i, 0)),
        ],
        out_shape=[
            jax.ShapeDtypeStruct((N, F), jnp.float32),
            jax.ShapeDtypeStruct((N, F), jnp.float32),
        ],
        interpret=interpret,
    )(node_feats, node_attrs, W_up, W_skipT)


def _edge_pre_body(ef_ref, ea_ref, s0_ref, wr1_ref, wr2_ref, wr3_ref, wr4_ref, m4_ref):
    x = ef_ref[...]
    w = _silu(jnp.dot(x, wr1_ref[...], preferred_element_type=jnp.float32) * (8.0 ** -0.5))
    w = _silu(jnp.dot(w, wr2_ref[...], preferred_element_type=jnp.float32) * (64.0 ** -0.5))
    w = _silu(jnp.dot(w, wr3_ref[...], preferred_element_type=jnp.float32) * (64.0 ** -0.5))
    w = jnp.dot(w, wr4_ref[...], preferred_element_type=jnp.float32) * (64.0 ** -0.5)
    s = s0_ref[...]
    y = ea_ref[...]
    m4_ref[0] = w[:, :F] * s * y[:, 0:1]
    b = w[:, F:] * s
    m4_ref[1] = b * y[:, 1:2]
    m4_ref[2] = b * y[:, 2:3]
    m4_ref[3] = b * y[:, 3:4]


def _edge_pre(edge_feats, edge_attrs, s0, W_r1, W_r2, W_r3, W_r4b, *, interpret=False):
    E = edge_feats.shape[0]
    EB = 2000
    return pl.pallas_call(
        _edge_pre_body,
        grid=(E // EB,),
        in_specs=[
            pl.BlockSpec((EB, 8), lambda i: (i, 0)),
            pl.BlockSpec((EB, 4), lambda i: (i, 0)),
            pl.BlockSpec((EB, F), lambda i: (i, 0)),
            pl.BlockSpec((8, 64), lambda i: (0, 0)),
            pl.BlockSpec((64, 64), lambda i: (0, 0)),
            pl.BlockSpec((64, 64), lambda i: (0, 0)),
            pl.BlockSpec((64, 2 * F), lambda i: (0, 0)),
        ],
        out_specs=pl.BlockSpec((4, EB, F), lambda i: (0, i, 0)),
        out_shape=jax.ShapeDtypeStruct((4, E, F), jnp.float32),
        interpret=interpret,
    )(edge_feats, edge_attrs, s0, W_r1, W_r2, W_r3, W_r4b)


def _node_post_body(
    p1_ref, p2_ref, na_ref, wl0_ref, wl1_ref, wm0T_ref, wm1T_ref, wo0_ref,
    wo1_ref, f0_ref, fx_ref, fy_ref, fz_ref,
):
    c1 = ((2 * F) ** -0.5) * 0.25
    c2 = (F * S) ** -0.5
    c3 = F ** -0.5
    a = na_ref[...]

    def species_tp(t, wT_ref):
        acc = jnp.zeros(t.shape, jnp.float32)
        for v in range(S):
            acc = acc + a[:, v : v + 1] * jnp.dot(
                t, wT_ref[v], preferred_element_type=jnp.float32
            )
        return acc * c2

    m0 = p1_ref[0] + p2_ref[0]
    t0 = jnp.dot(m0, wl0_ref[...], preferred_element_type=jnp.float32) * c1
    o0 = species_tp(t0, wm0T_ref)
    f0_ref[...] = jnp.dot(o0, wo0_ref[...], preferred_element_type=jnp.float32) * c3
    for i, out_ref in ((1, fx_ref), (2, fy_ref), (3, fz_ref)):
        mi = p1_ref[i] + p2_ref[i]
        t = jnp.dot(mi, wl1_ref[...], preferred_element_type=jnp.float32) * c1
        o = species_tp(t, wm1T_ref)
        out_ref[...] = jnp.dot(o, wo1_ref[...], preferred_element_type=jnp.float32) * c3


def _node_post(p1, p2, node_attrs, W_lin0f, W_lin1f, W_msg0T, W_msg1T, W_out0,
               W_out1, *, interpret=False):
    N = node_attrs.shape[0]
    NB = 1000
    return pl.pallas_call(
        _node_post_body,
        grid=(N // NB,),
        in_specs=[
            pl.BlockSpec((4, NB, F), lambda i: (0, i, 0)),
            pl.BlockSpec((4, NB, F), lambda i: (0, i, 0)),
            pl.BlockSpec((NB, S), lambda i: (i, 0)),
            pl.BlockSpec((F, F), lambda i: (0, 0)),
            pl.BlockSpec((F, F), lambda i: (0, 0)),
            pl.BlockSpec((S, F, F), lambda i: (0, 0, 0)),
            pl.BlockSpec((S, F, F), lambda i: (0, 0, 0)),
            pl.BlockSpec((F, F), lambda i: (0, 0)),
            pl.BlockSpec((F, F), lambda i: (0, 0)),
        ],
        out_specs=[pl.BlockSpec((NB, F), lambda i: (i, 0)) for _ in range(4)],
        out_shape=[jax.ShapeDtypeStruct((N, F), jnp.float32) for _ in range(4)],
        interpret=interpret,
    )(p1, p2, node_attrs, W_lin0f, W_lin1f, W_msg0T, W_msg1T, W_out0, W_out1)


def _sc_gather(h0, snd, lo, EH):
    N = h0.shape[0]
    NW = 32
    e_per_w = EH // NW
    CH = 200
    mesh = plsc.VectorSubcoreMesh(core_axis_name="c", subcore_axis_name="s")

    NCH = e_per_w // CH

    @functools.partial(
        pl.kernel,
        out_type=jax.ShapeDtypeStruct((EH, F), jnp.float32),
        mesh=mesh,
        scratch_types=[
            pltpu.VMEM((CH,), jnp.int32),
            pltpu.VMEM((CH,), jnp.int32),
            pltpu.VMEM((2, CH, F), jnp.float32),
            pltpu.SemaphoreType.DMA,
            pltpu.SemaphoreType.DMA,
            pltpu.SemaphoreType.DMA,
            pltpu.SemaphoreType.DMA,
        ],
    )
    def gk(h0_hbm, snd_hbm, out_hbm, i_v0, i_v1, rows_v, is0, is1, gs0, gs1):
        wid = lax.axis_index("s") * 2 + lax.axis_index("c")
        base = wid * e_per_w
        idxb = (i_v0, i_v1)
        isems = (is0, is1)
        gsems = (gs0, gs1)

        def idx_issue(i, b):
            pltpu.async_copy(
                snd_hbm.at[pl.ds(lo + base + i * CH, CH)], idxb[b], isems[b]
            )

        def idx_wait(i, b):
            pltpu.make_async_copy(
                snd_hbm.at[pl.ds(lo + base + i * CH, CH)], idxb[b], isems[b]
            ).wait()

        def g_issue(b):
            pltpu.async_copy(h0_hbm.at[idxb[b]], rows_v.at[b], gsems[b])

        def g_wait(b):
            pltpu.make_async_copy(h0_hbm.at[idxb[b]], rows_v.at[b], gsems[b]).wait()

        def step(i, bi):
            idx_wait(i + 1, 1 - bi)
            g_issue(1 - bi)
            g_wait(bi)
            pltpu.sync_copy(rows_v.at[bi], out_hbm.at[pl.ds(base + i * CH, CH)])

        idx_issue(0, 0)
        idx_wait(0, 0)
        g_issue(0)
        idx_issue(1, 1)

        def pair_body(i2, carry):
            for bi in range(2):
                i = i2 * 2 + bi
                step(i, bi)

                @pl.when(i + 2 < NCH)
                def _():
                    idx_issue(i + 2, bi)

            return carry

        P2 = 2 * ((NCH - 1) // 2)
        lax.fori_loop(0, (NCH - 1) // 2, pair_body, 0)
        for i in range(P2, NCH - 1):
            step(i, i % 2)
            if i + 2 < NCH:
                idx_issue(i + 2, i % 2)
        bl = (NCH - 1) % 2
        g_wait(bl)
        pltpu.sync_copy(
            rows_v.at[bl], out_hbm.at[pl.ds(base + (NCH - 1) * CH, CH)]
        )

    return gk(h0, snd)


def _sc_scatter(m4, rcv, lo, EH, init4):
    Np = init4.shape[1]
    CH = 80
    n_per_tile = Np // 16
    e_per_tile = EH // 16
    mesh = plsc.VectorSubcoreMesh(core_axis_name="c", subcore_axis_name="s")

    NCH = e_per_tile // CH

    @functools.partial(
        pl.kernel,
        out_type=jax.ShapeDtypeStruct((4, Np, F), jnp.float32),
        mesh=mesh,
        scratch_types=[
            pltpu.VMEM_SHARED((Np, F), jnp.float32),
            pltpu.VMEM((CH,), jnp.int32),
            pltpu.VMEM((CH,), jnp.int32),
            pltpu.VMEM((2, CH, F), jnp.float32),
            pltpu.SemaphoreType.DMA,
            pltpu.SemaphoreType.DMA,
        ],
    )
    def sk(m4_hbm, rcv_hbm, init4_hbm, out_hbm, acc_sh, i_v0, i_v1, m_v,
           ls0, ls1):
        c = lax.axis_index("c")
        s = lax.axis_index("s")
        r0 = s * n_per_tile
        idxb = (i_v0, i_v1)
        lsems = (ls0, ls1)
        for j in range(2):
            p = c * 2 + j

            def ld_pairs(i, b):
                off = s * e_per_tile + i * CH
                return (
                    (rcv_hbm.at[pl.ds(lo + off, CH)], idxb[b]),
                    (m4_hbm.at[p].at[pl.ds(off, CH)], m_v.at[b]),
                )

            def ld_issue(i, b):
                for sr, dr in ld_pairs(i, b):
                    pltpu.async_copy(sr, dr, lsems[b])

            def ld_wait(i, b):
                for sr, dr in ld_pairs(i, b):
                    pltpu.make_async_copy(sr, dr, lsems[b]).wait()

            def step(i, bi):
                ld_wait(i, bi)
                pltpu.sync_copy(m_v.at[bi], acc_sh.at[idxb[bi]], add=True)

            pltpu.sync_copy(
                init4_hbm.at[p].at[pl.ds(r0, n_per_tile)],
                acc_sh.at[pl.ds(r0, n_per_tile)],
            )
            plsc.subcore_barrier()

            ld_issue(0, 0)
            ld_issue(1, 1)

            def pair_body(i2, carry):
                for bi in range(2):
                    i = i2 * 2 + bi
                    step(i, bi)

                    @pl.when(i + 2 < NCH)
                    def _():
                        ld_issue(i + 2, bi)

                return carry

            P2 = 2 * (NCH // 2)
            lax.fori_loop(0, NCH // 2, pair_body, 0)
            for i in range(P2, NCH):
                step(i, i % 2)
            plsc.subcore_barrier()
            pltpu.sync_copy(
                acc_sh.at[pl.ds(r0, n_per_tile)],
                out_hbm.at[p].at[pl.ds(r0, n_per_tile)],
            )
            plsc.subcore_barrier()

    return sk(m4, rcv, init4)


def _half(x, lo, n):
    return lax.slice(x, (lo, 0), (lo + n, x.shape[1]))


def kernel(node_attrs, node_feats, edge_attrs, edge_feats, W_skip, W_up, W_r1,
           W_r2, W_r3, W_r4, W_lin0, W_lin1, W_msg0, W_msg1, W_out0, W_out1,
           edge_index, species):
    N = node_feats.shape[0]
    snd = edge_index[0]
    rcv = edge_index[1]

    h0, skip0 = _node_pre(node_feats, node_attrs, W_up, W_skip.transpose(1, 0, 2))
    Np = ((N // 16 + 7) // 8 * 8) * 16
    E = snd.shape[0]
    EH = E // 2
    W_r4b = W_r4[:, : 2 * F]
    zeros4 = jnp.zeros((4, Np, F), jnp.float32)
    parts = []
    for lo in (0, EH):
        s0_h = _sc_gather(h0, snd, lo, EH)
        m4_h = _edge_pre(
            _half(edge_feats, lo, EH), _half(edge_attrs, lo, EH),
            s0_h, W_r1, W_r2, W_r3, W_r4b,
        )
        parts.append(_sc_scatter(m4_h, rcv, lo, EH, zeros4))
    f0, fx, fy, fz = _node_post(
        parts[0], parts[1], node_attrs, W_lin0[:F], W_lin1[:F],
        W_msg0.transpose(1, 0, 2), W_msg1.transpose(1, 0, 2), W_out0, W_out1,
    )
    message = jnp.concatenate(
        [f0, jnp.stack([fx, fy, fz], axis=-1).reshape(N, 3 * F)], axis=1
    )
    skip = jnp.concatenate([skip0, jnp.zeros((N, 3 * F), jnp.float32)], axis=1)
    return message, skip


# speedup vs baseline: 3.3021x; 1.0018x over previous
import functools

import jax
import jax.numpy as jnp
from jax import lax
from jax.experimental import pallas as pl
from jax.experimental.pallas import tpu as pltpu
from jax.experimental.pallas import tpu_sc as plsc

F = 128
S = 10


def _silu(x):
    return x * (1.0 / (1.0 + jnp.exp(-x)))


def _node_pre_body(nf_ref, na_ref, wup_ref, wskipT_ref, h0_ref, skip0_ref):
    x = nf_ref[...]
    h0_ref[...] = jnp.dot(x, wup_ref[...], preferred_element_type=jnp.float32) * (
        F ** -0.5
    )
    a = na_ref[...]
    acc = jnp.zeros(x.shape, jnp.float32)
    for v in range(S):
        acc = acc + a[:, v : v + 1] * jnp.dot(
            x, wskipT_ref[v], preferred_element_type=jnp.float32
        )
    skip0_ref[...] = acc * ((F * S) ** -0.5)


def _node_pre(node_feats, node_attrs, W_up, W_skipT, *, interpret=False):
    N = node_feats.shape[0]
    NB = 1000
    return pl.pallas_call(
        _node_pre_body,
        grid=(N // NB,),
        in_specs=[
            pl.BlockSpec((NB, F), lambda i: (i, 0)),
            pl.BlockSpec((NB, S), lambda i: (i, 0)),
            pl.BlockSpec((F, F), lambda i: (0, 0)),
            pl.BlockSpec((S, F, F), lambda i: (0, 0, 0)),
        ],
        out_specs=[
            pl.BlockSpec((NB, F), lambda i: (i, 0)),
            pl.BlockSpec((NB, F), lambda i: (i, 0)),
        ],
        out_shape=[
            jax.ShapeDtypeStruct((N, F), jnp.float32),
            jax.ShapeDtypeStruct((N, F), jnp.float32),
        ],
        interpret=interpret,
    )(node_feats, node_attrs, W_up, W_skipT)


def _edge_pre_body(ef_ref, ea_ref, s0_ref, wr1_ref, wr2_ref, wr3_ref, wr4_ref, m4_ref):
    x = ef_ref[...]
    w = _silu(jnp.dot(x, wr1_ref[...], preferred_element_type=jnp.float32) * (8.0 ** -0.5))
    w = _silu(jnp.dot(w, wr2_ref[...], preferred_element_type=jnp.float32) * (64.0 ** -0.5))
    w = _silu(jnp.dot(w, wr3_ref[...], preferred_element_type=jnp.float32) * (64.0 ** -0.5))
    w = jnp.dot(w, wr4_ref[...], preferred_element_type=jnp.float32) * (64.0 ** -0.5)
    s = s0_ref[...]
    y = ea_ref[...]
    m4_ref[0] = w[:, :F] * s * y[:, 0:1]
    b = w[:, F:] * s
    m4_ref[1] = b * y[:, 1:2]
    m4_ref[2] = b * y[:, 2:3]
    m4_ref[3] = b * y[:, 3:4]


def _edge_pre(edge_feats, edge_attrs, s0, W_r1, W_r2, W_r3, W_r4b, *, interpret=False):
    E = edge_feats.shape[0]
    EB = 4000
    return pl.pallas_call(
        _edge_pre_body,
        grid=(E // EB,),
        in_specs=[
            pl.BlockSpec((EB, 8), lambda i: (i, 0)),
            pl.BlockSpec((EB, 4), lambda i: (i, 0)),
            pl.BlockSpec((EB, F), lambda i: (i, 0)),
            pl.BlockSpec((8, 64), lambda i: (0, 0)),
            pl.BlockSpec((64, 64), lambda i: (0, 0)),
            pl.BlockSpec((64, 64), lambda i: (0, 0)),
            pl.BlockSpec((64, 2 * F), lambda i: (0, 0)),
        ],
        out_specs=pl.BlockSpec((4, EB, F), lambda i: (0, i, 0)),
        out_shape=jax.ShapeDtypeStruct((4, E, F), jnp.float32),
        interpret=interpret,
    )(edge_feats, edge_attrs, s0, W_r1, W_r2, W_r3, W_r4b)


def _node_post_body(
    p1_ref, p2_ref, na_ref, wl0_ref, wl1_ref, wm0T_ref, wm1T_ref, wo0_ref,
    wo1_ref, f0_ref, fx_ref, fy_ref, fz_ref,
):
    c1 = ((2 * F) ** -0.5) * 0.25
    c2 = (F * S) ** -0.5
    c3 = F ** -0.5
    a = na_ref[...]

    def species_tp(t, wT_ref):
        acc = jnp.zeros(t.shape, jnp.float32)
        for v in range(S):
            acc = acc + a[:, v : v + 1] * jnp.dot(
                t, wT_ref[v], preferred_element_type=jnp.float32
            )
        return acc * c2

    m0 = p1_ref[0] + p2_ref[0]
    t0 = jnp.dot(m0, wl0_ref[...], preferred_element_type=jnp.float32) * c1
    o0 = species_tp(t0, wm0T_ref)
    f0_ref[...] = jnp.dot(o0, wo0_ref[...], preferred_element_type=jnp.float32) * c3
    for i, out_ref in ((1, fx_ref), (2, fy_ref), (3, fz_ref)):
        mi = p1_ref[i] + p2_ref[i]
        t = jnp.dot(mi, wl1_ref[...], preferred_element_type=jnp.float32) * c1
        o = species_tp(t, wm1T_ref)
        out_ref[...] = jnp.dot(o, wo1_ref[...], preferred_element_type=jnp.float32) * c3


def _node_post(p1, p2, node_attrs, W_lin0f, W_lin1f, W_msg0T, W_msg1T, W_out0,
               W_out1, *, interpret=False):
    N = node_attrs.shape[0]
    NB = 1000
    return pl.pallas_call(
        _node_post_body,
        grid=(N // NB,),
        in_specs=[
            pl.BlockSpec((4, NB, F), lambda i: (0, i, 0)),
            pl.BlockSpec((4, NB, F), lambda i: (0, i, 0)),
            pl.BlockSpec((NB, S), lambda i: (i, 0)),
            pl.BlockSpec((F, F), lambda i: (0, 0)),
            pl.BlockSpec((F, F), lambda i: (0, 0)),
            pl.BlockSpec((S, F, F), lambda i: (0, 0, 0)),
            pl.BlockSpec((S, F, F), lambda i: (0, 0, 0)),
            pl.BlockSpec((F, F), lambda i: (0, 0)),
            pl.BlockSpec((F, F), lambda i: (0, 0)),
        ],
        out_specs=[pl.BlockSpec((NB, F), lambda i: (i, 0)) for _ in range(4)],
        out_shape=[jax.ShapeDtypeStruct((N, F), jnp.float32) for _ in range(4)],
        interpret=interpret,
    )(p1, p2, node_attrs, W_lin0f, W_lin1f, W_msg0T, W_msg1T, W_out0, W_out1)


def _sc_gather(h0, snd, lo, EH):
    N = h0.shape[0]
    NW = 32
    e_per_w = EH // NW
    CH = 200
    mesh = plsc.VectorSubcoreMesh(core_axis_name="c", subcore_axis_name="s")

    NCH = e_per_w // CH

    @functools.partial(
        pl.kernel,
        out_type=jax.ShapeDtypeStruct((EH, F), jnp.float32),
        mesh=mesh,
        scratch_types=[
            pltpu.VMEM((CH,), jnp.int32),
            pltpu.VMEM((CH,), jnp.int32),
            pltpu.VMEM((2, CH, F), jnp.float32),
            pltpu.SemaphoreType.DMA,
            pltpu.SemaphoreType.DMA,
            pltpu.SemaphoreType.DMA,
            pltpu.SemaphoreType.DMA,
        ],
    )
    def gk(h0_hbm, snd_hbm, out_hbm, i_v0, i_v1, rows_v, is0, is1, gs0, gs1):
        wid = lax.axis_index("s") * 2 + lax.axis_index("c")
        base = wid * e_per_w
        idxb = (i_v0, i_v1)
        isems = (is0, is1)
        gsems = (gs0, gs1)

        def idx_issue(i, b):
            pltpu.async_copy(
                snd_hbm.at[pl.ds(lo + base + i * CH, CH)], idxb[b], isems[b]
            )

        def idx_wait(i, b):
            pltpu.make_async_copy(
                snd_hbm.at[pl.ds(lo + base + i * CH, CH)], idxb[b], isems[b]
            ).wait()

        def g_issue(b):
            pltpu.async_copy(h0_hbm.at[idxb[b]], rows_v.at[b], gsems[b])

        def g_wait(b):
            pltpu.make_async_copy(h0_hbm.at[idxb[b]], rows_v.at[b], gsems[b]).wait()

        def step(i, bi):
            idx_wait(i + 1, 1 - bi)
            g_issue(1 - bi)
            g_wait(bi)
            pltpu.sync_copy(rows_v.at[bi], out_hbm.at[pl.ds(base + i * CH, CH)])

        idx_issue(0, 0)
        idx_wait(0, 0)
        g_issue(0)
        idx_issue(1, 1)

        def pair_body(i2, carry):
            for bi in range(2):
                i = i2 * 2 + bi
                step(i, bi)

                @pl.when(i + 2 < NCH)
                def _():
                    idx_issue(i + 2, bi)

            return carry

        P2 = 2 * ((NCH - 1) // 2)
        lax.fori_loop(0, (NCH - 1) // 2, pair_body, 0)
        for i in range(P2, NCH - 1):
            step(i, i % 2)
            if i + 2 < NCH:
                idx_issue(i + 2, i % 2)
        bl = (NCH - 1) % 2
        g_wait(bl)
        pltpu.sync_copy(
            rows_v.at[bl], out_hbm.at[pl.ds(base + (NCH - 1) * CH, CH)]
        )

    return gk(h0, snd)


def _sc_scatter(m4, rcv, lo, EH, init4):
    Np = init4.shape[1]
    CH = 80
    n_per_tile = Np // 16
    e_per_tile = EH // 16
    mesh = plsc.VectorSubcoreMesh(core_axis_name="c", subcore_axis_name="s")

    NCH = e_per_tile // CH

    @functools.partial(
        pl.kernel,
        out_type=jax.ShapeDtypeStruct((4, Np, F), jnp.float32),
        mesh=mesh,
        scratch_types=[
            pltpu.VMEM_SHARED((Np, F), jnp.float32),
            pltpu.VMEM((CH,), jnp.int32),
            pltpu.VMEM((CH,), jnp.int32),
            pltpu.VMEM((2, CH, F), jnp.float32),
            pltpu.SemaphoreType.DMA,
            pltpu.SemaphoreType.DMA,
        ],
    )
    def sk(m4_hbm, rcv_hbm, init4_hbm, out_hbm, acc_sh, i_v0, i_v1, m_v,
           ls0, ls1):
        c = lax.axis_index("c")
        s = lax.axis_index("s")
        r0 = s * n_per_tile
        idxb = (i_v0, i_v1)
        lsems = (ls0, ls1)
        for j in range(2):
            p = c * 2 + j

            def ld_pairs(i, b):
                off = s * e_per_tile + i * CH
                return (
                    (rcv_hbm.at[pl.ds(lo + off, CH)], idxb[b]),
                    (m4_hbm.at[p].at[pl.ds(off, CH)], m_v.at[b]),
                )

            def ld_issue(i, b):
                for sr, dr in ld_pairs(i, b):
                    pltpu.async_copy(sr, dr, lsems[b])

            def ld_wait(i, b):
                for sr, dr in ld_pairs(i, b):
                    pltpu.make_async_copy(sr, dr, lsems[b]).wait()

            def step(i, bi):
                ld_wait(i, bi)
                pltpu.sync_copy(m_v.at[bi], acc_sh.at[idxb[bi]], add=True)

            pltpu.sync_copy(
                init4_hbm.at[p].at[pl.ds(r0, n_per_tile)],
                acc_sh.at[pl.ds(r0, n_per_tile)],
            )
            plsc.subcore_barrier()

            ld_issue(0, 0)
            ld_issue(1, 1)

            def pair_body(i2, carry):
                for bi in range(2):
                    i = i2 * 2 + bi
                    step(i, bi)

                    @pl.when(i + 2 < NCH)
                    def _():
                        ld_issue(i + 2, bi)

                return carry

            P2 = 2 * (NCH // 2)
            lax.fori_loop(0, NCH // 2, pair_body, 0)
            for i in range(P2, NCH):
                step(i, i % 2)
            plsc.subcore_barrier()
            pltpu.sync_copy(
                acc_sh.at[pl.ds(r0, n_per_tile)],
                out_hbm.at[p].at[pl.ds(r0, n_per_tile)],
            )
            plsc.subcore_barrier()

    return sk(m4, rcv, init4)


def _half(x, lo, n):
    return lax.slice(x, (lo, 0), (lo + n, x.shape[1]))


def kernel(node_attrs, node_feats, edge_attrs, edge_feats, W_skip, W_up, W_r1,
           W_r2, W_r3, W_r4, W_lin0, W_lin1, W_msg0, W_msg1, W_out0, W_out1,
           edge_index, species):
    N = node_feats.shape[0]
    snd = edge_index[0]
    rcv = edge_index[1]

    h0, skip0 = _node_pre(node_feats, node_attrs, W_up, W_skip.transpose(1, 0, 2))
    Np = ((N // 16 + 7) // 8 * 8) * 16
    E = snd.shape[0]
    EH = E // 2
    W_r4b = W_r4[:, : 2 * F]
    zeros4 = jnp.zeros((4, Np, F), jnp.float32)
    parts = []
    for lo in (0, EH):
        s0_h = _sc_gather(h0, snd, lo, EH)
        m4_h = _edge_pre(
            _half(edge_feats, lo, EH), _half(edge_attrs, lo, EH),
            s0_h, W_r1, W_r2, W_r3, W_r4b,
        )
        parts.append(_sc_scatter(m4_h, rcv, lo, EH, zeros4))
    f0, fx, fy, fz = _node_post(
        parts[0], parts[1], node_attrs, W_lin0[:F], W_lin1[:F],
        W_msg0.transpose(1, 0, 2), W_msg1.transpose(1, 0, 2), W_out0, W_out1,
    )
    message = jnp.concatenate(
        [f0, jnp.stack([fx, fy, fz], axis=-1).reshape(N, 3 * F)], axis=1
    )
    skip = jnp.concatenate([skip0, jnp.zeros((N, 3 * F), jnp.float32)], axis=1)
    return message, skip
